# Initial kernel scaffold; baseline (speedup 1.0000x reference)
import jax
import jax.numpy as jnp
from jax.experimental import pallas as pl


def kernel(input, h_positions, v_positions):
    raise NotImplementedError("write your pallas kernel here")


# speedup vs baseline: 3.7444x; 3.7444x over previous
import jax
import jax.numpy as jnp
from jax import lax
from jax.experimental import pallas as pl
from jax.experimental.pallas import tpu as pltpu

H = 384
W = 384
C = 192
NPOS = 31
NSEG = NPOS + 1
HB = 64
IB = 32


def _reduce_rows_kernel(hp_ref, x_ref, s1_ref, ridx_ref):
    h = pl.program_id(0)
    col_i = lax.broadcasted_iota(jnp.int32, (HB, 1), 0) + h * HB
    row_i = lax.broadcasted_iota(jnp.int32, (1, HB), 1) + h * HB
    acc_c = jnp.zeros((HB, 1), jnp.int32)
    acc_r = jnp.zeros((1, HB), jnp.int32)
    for k in range(NPOS):
        p = hp_ref[0, k]
        acc_c += (p <= col_i).astype(jnp.int32)
        acc_r += (p <= row_i).astype(jnp.int32)
    ridx_ref[...] = acc_c
    onehot_t = (acc_r == lax.broadcasted_iota(jnp.int32, (NSEG, HB), 0)
                ).astype(jnp.float32)
    part = lax.dot_general(onehot_t, x_ref[...], (((1,), (0,)), ((), ())),
                           preferred_element_type=jnp.float32)

    @pl.when(h == 0)
    def _():
        s1_ref[...] = part

    @pl.when(h > 0)
    def _():
        s1_ref[...] += part


def _reduce_cols_kernel(vp_ref, s1_ref, means_ref, onehotc_ref):
    jj_r = lax.broadcasted_iota(jnp.int32, (1, W), 1)
    jj_c = lax.broadcasted_iota(jnp.int32, (W, 1), 0)
    acc_r = jnp.zeros((1, W), jnp.int32)
    acc_c = jnp.zeros((W, 1), jnp.int32)
    for k in range(NPOS):
        p = vp_ref[0, k]
        acc_r += (p <= jj_r).astype(jnp.int32)
        acc_c += (p <= jj_c).astype(jnp.int32)
    oh_t = (acc_r == lax.broadcasted_iota(jnp.int32, (NSEG, W), 0)
            ).astype(jnp.float32)
    onehotc_ref[...] = (acc_c == lax.broadcasted_iota(jnp.int32, (W, NSEG), 1)
                        ).astype(jnp.float32)
    cnt = jnp.sum(oh_t, axis=1, keepdims=True)
    oh_s = oh_t * (1.0 / jnp.maximum(cnt, 1.0))
    for r in range(NSEG):
        slab = s1_ref[r]
        means_ref[r] = lax.dot_general(oh_s, slab, (((1,), (0,)), ((), ())),
                                       preferred_element_type=jnp.float32)


def _expand_kernel(hp_ref, ridx_ref, means_ref, onehotc_ref, out_ref):
    oh = onehotc_ref[...]
    base = pl.program_id(0) * IB

    def body(ii, carry):
        r = ridx_ref[base + ii, 0]
        lo = jnp.where(r > 0, hp_ref[0, jnp.maximum(r - 1, 0)], 0)
        hi = jnp.where(r < NPOS, hp_ref[0, jnp.minimum(r, NPOS - 1)], H)
        rs = 1.0 / (hi - lo).astype(jnp.float32)
        m = means_ref[pl.ds(r, 1)][0]
        row = lax.dot_general(oh, m, (((1,), (0,)), ((), ())),
                              preferred_element_type=jnp.float32) * rs
        out_ref[pl.ds(ii, 1)] = row[None]
        return carry

    lax.fori_loop(0, IB, body, 0)


def kernel(input, h_positions, v_positions):
    x = input.reshape(H, W * C)
    hp = h_positions.astype(jnp.int32).reshape(1, NPOS)
    vp = v_positions.astype(jnp.int32).reshape(1, NPOS)

    s1, ridx = pl.pallas_call(
        _reduce_rows_kernel,
        grid=(H // HB,),
        in_specs=[
            pl.BlockSpec(memory_space=pltpu.SMEM),
            pl.BlockSpec((HB, W * C), lambda h: (h, 0)),
        ],
        out_specs=[
            pl.BlockSpec((NSEG, W * C), lambda h: (0, 0)),
            pl.BlockSpec((HB, 1), lambda h: (h, 0)),
        ],
        out_shape=[
            jax.ShapeDtypeStruct((NSEG, W * C), jnp.float32),
            jax.ShapeDtypeStruct((H, 1), jnp.int32),
        ],
    )(hp, x)

    means, onehotc = pl.pallas_call(
        _reduce_cols_kernel,
        in_specs=[
            pl.BlockSpec(memory_space=pltpu.SMEM),
            pl.BlockSpec((NSEG, W, C), lambda: (0, 0, 0)),
        ],
        out_shape=[
            jax.ShapeDtypeStruct((NSEG, NSEG, C), jnp.float32),
            jax.ShapeDtypeStruct((W, NSEG), jnp.float32),
        ],
    )(vp, s1.reshape(NSEG, W, C))

    y = pl.pallas_call(
        _expand_kernel,
        grid=(H // IB,),
        in_specs=[
            pl.BlockSpec(memory_space=pltpu.SMEM),
            pl.BlockSpec(memory_space=pltpu.SMEM),
            pl.BlockSpec((NSEG, NSEG, C), lambda h: (0, 0, 0)),
            pl.BlockSpec((W, NSEG), lambda h: (0, 0)),
        ],
        out_specs=pl.BlockSpec((IB, W, C), lambda h: (h, 0, 0)),
        out_shape=jax.ShapeDtypeStruct((H, W, C), jnp.float32),
    )(hp, ridx, means, onehotc)

    return y[None]


# speedup vs baseline: 3.9428x; 1.0530x over previous
import jax
import jax.numpy as jnp
from jax import lax
from jax.experimental import pallas as pl
from jax.experimental.pallas import tpu as pltpu

H = 384
W = 384
C = 192
NPOS = 31
NSEG = NPOS + 1
HB = 32
IB = 32


def _reduce_rows_kernel(hp_ref, x_ref, s1_ref, ridx_ref):
    h = pl.program_id(0)
    col_i = lax.broadcasted_iota(jnp.int32, (HB, 1), 0) + h * HB
    row_i = lax.broadcasted_iota(jnp.int32, (1, HB), 1) + h * HB
    acc_c = jnp.zeros((HB, 1), jnp.int32)
    acc_r = jnp.zeros((1, HB), jnp.int32)
    for k in range(NPOS):
        p = hp_ref[0, k]
        acc_c += (p <= col_i).astype(jnp.int32)
        acc_r += (p <= row_i).astype(jnp.int32)
    ridx_ref[...] = acc_c
    onehot_t = (acc_r == lax.broadcasted_iota(jnp.int32, (NSEG, HB), 0)
                ).astype(jnp.float32)
    part = lax.dot_general(onehot_t, x_ref[...], (((1,), (0,)), ((), ())),
                           preferred_element_type=jnp.float32)

    @pl.when(h == 0)
    def _():
        s1_ref[...] = part

    @pl.when(h > 0)
    def _():
        s1_ref[...] += part


def _reduce_cols_kernel(vp_ref, s1_ref, means_ref, onehotc_ref):
    jj_r = lax.broadcasted_iota(jnp.int32, (1, W), 1)
    jj_c = lax.broadcasted_iota(jnp.int32, (W, 1), 0)
    acc_r = jnp.zeros((1, W), jnp.int32)
    acc_c = jnp.zeros((W, 1), jnp.int32)
    for k in range(NPOS):
        p = vp_ref[0, k]
        acc_r += (p <= jj_r).astype(jnp.int32)
        acc_c += (p <= jj_c).astype(jnp.int32)
    oh_t = (acc_r == lax.broadcasted_iota(jnp.int32, (NSEG, W), 0)
            ).astype(jnp.float32)
    onehotc_ref[...] = (acc_c == lax.broadcasted_iota(jnp.int32, (W, NSEG), 1)
                        ).astype(jnp.float32)
    cnt = jnp.sum(oh_t, axis=1, keepdims=True)
    oh_s = oh_t * (1.0 / jnp.maximum(cnt, 1.0))
    for r in range(NSEG):
        slab = s1_ref[r]
        means_ref[r] = lax.dot_general(oh_s, slab, (((1,), (0,)), ((), ())),
                                       preferred_element_type=jnp.float32)


def _expand_kernel(hp_ref, ridx_ref, means_ref, onehotc_ref, out_ref):
    oh = onehotc_ref[...]
    base = pl.program_id(0) * IB

    def body(ii, carry):
        r = ridx_ref[base + ii, 0]
        lo = jnp.where(r > 0, hp_ref[0, jnp.maximum(r - 1, 0)], 0)
        hi = jnp.where(r < NPOS, hp_ref[0, jnp.minimum(r, NPOS - 1)], H)
        rs = 1.0 / (hi - lo).astype(jnp.float32)
        m = means_ref[pl.ds(r, 1)][0]
        row = lax.dot_general(oh, m, (((1,), (0,)), ((), ())),
                              preferred_element_type=jnp.float32) * rs
        out_ref[pl.ds(ii, 1)] = row[None]
        return carry

    lax.fori_loop(0, IB, body, 0)


def kernel(input, h_positions, v_positions):
    x = input[0]
    hp = h_positions.astype(jnp.int32).reshape(1, NPOS)
    vp = v_positions.astype(jnp.int32).reshape(1, NPOS)

    s1, ridx = pl.pallas_call(
        _reduce_rows_kernel,
        grid=(H // HB,),
        in_specs=[
            pl.BlockSpec(memory_space=pltpu.SMEM),
            pl.BlockSpec((HB, W, C), lambda h: (h, 0, 0)),
        ],
        out_specs=[
            pl.BlockSpec((NSEG, W, C), lambda h: (0, 0, 0)),
            pl.BlockSpec((HB, 1), lambda h: (h, 0)),
        ],
        out_shape=[
            jax.ShapeDtypeStruct((NSEG, W, C), jnp.float32),
            jax.ShapeDtypeStruct((H, 1), jnp.int32),
        ],
    )(hp, x)

    means, onehotc = pl.pallas_call(
        _reduce_cols_kernel,
        in_specs=[
            pl.BlockSpec(memory_space=pltpu.SMEM),
            pl.BlockSpec((NSEG, W, C), lambda: (0, 0, 0)),
        ],
        out_shape=[
            jax.ShapeDtypeStruct((NSEG, NSEG, C), jnp.float32),
            jax.ShapeDtypeStruct((W, NSEG), jnp.float32),
        ],
    )(vp, s1)

    y = pl.pallas_call(
        _expand_kernel,
        grid=(H // IB,),
        in_specs=[
            pl.BlockSpec(memory_space=pltpu.SMEM),
            pl.BlockSpec(memory_space=pltpu.SMEM),
            pl.BlockSpec((NSEG, NSEG, C), lambda h: (0, 0, 0)),
            pl.BlockSpec((W, NSEG), lambda h: (0, 0)),
        ],
        out_specs=pl.BlockSpec((IB, W, C), lambda h: (h, 0, 0)),
        out_shape=jax.ShapeDtypeStruct((H, W, C), jnp.float32),
    )(hp, ridx, means, onehotc)

    return y[None]


# speedup vs baseline: 10.8421x; 2.7498x over previous
import jax
import jax.numpy as jnp
from jax import lax
from jax.experimental import pallas as pl
from jax.experimental.pallas import tpu as pltpu

H = 384
W = 384
C = 192
NPOS = 31
NSEG = NPOS + 1
HB = 32
IB = 32


def _reduce_kernel(hp_ref, vp_ref, x_ref, means_ref, ohct_ref, ridx_ref,
                   s1_ref):
    h = pl.program_id(0)
    nsteps = pl.num_programs(0)
    col_i = lax.broadcasted_iota(jnp.int32, (HB, 1), 0) + h * HB
    row_i = lax.broadcasted_iota(jnp.int32, (1, HB), 1) + h * HB
    acc_c = jnp.zeros((HB, 1), jnp.int32)
    acc_r = jnp.zeros((1, HB), jnp.int32)
    for k in range(NPOS):
        p = hp_ref[0, k]
        acc_c += (p <= col_i).astype(jnp.int32)
        acc_r += (p <= row_i).astype(jnp.int32)
    ridx_ref[...] = acc_c
    onehot_t = (acc_r == lax.broadcasted_iota(jnp.int32, (NSEG, HB), 0)
                ).astype(jnp.float32)
    part = lax.dot_general(onehot_t, x_ref[...], (((1,), (0,)), ((), ())),
                           preferred_element_type=jnp.float32)

    @pl.when(h == 0)
    def _():
        s1_ref[...] = part

    @pl.when(h > 0)
    def _():
        s1_ref[...] += part

    @pl.when(h == nsteps - 1)
    def _():
        jj_r = lax.broadcasted_iota(jnp.int32, (1, W), 1)
        jj_c = lax.broadcasted_iota(jnp.int32, (W, 1), 0)
        acc_jr = jnp.zeros((1, W), jnp.int32)
        acc_jc = jnp.zeros((W, 1), jnp.int32)
        for k in range(NPOS):
            p = vp_ref[0, k]
            acc_jr += (p <= jj_r).astype(jnp.int32)
            acc_jc += (p <= jj_c).astype(jnp.int32)
        ohct_ref[...] = (acc_jr == lax.broadcasted_iota(jnp.int32, (NSEG, W), 0)
                         ).astype(jnp.float32)
        ohc = (acc_jc == lax.broadcasted_iota(jnp.int32, (W, NSEG), 1)
               ).astype(jnp.float32)
        cnt = jnp.sum(ohc, axis=0, keepdims=True)
        ohc_s = ohc * (1.0 / jnp.maximum(cnt, 1.0))
        for r in range(NSEG):
            means_ref[r] = lax.dot_general(
                s1_ref[r], ohc_s, (((1,), (0,)), ((), ())),
                preferred_element_type=jnp.float32)


def _expand_kernel(hp_ref, ridx_ref, means_ref, ohct_ref, out_ref):
    ohct = ohct_ref[...]
    base = pl.program_id(0) * IB

    def body(ii, carry):
        r = ridx_ref[base + ii, 0]
        lo = jnp.where(r > 0, hp_ref[0, jnp.maximum(r - 1, 0)], 0)
        hi = jnp.where(r < NPOS, hp_ref[0, jnp.minimum(r, NPOS - 1)], H)
        rs = 1.0 / (hi - lo).astype(jnp.float32)
        m = means_ref[pl.ds(r, 1)][0]
        row = lax.dot_general(m, ohct, (((1,), (0,)), ((), ())),
                              preferred_element_type=jnp.float32) * rs
        out_ref[pl.ds(ii, 1)] = row[None]
        return carry

    lax.fori_loop(0, IB, body, 0)


def kernel(input, h_positions, v_positions):
    xt = jnp.transpose(input[0], (0, 2, 1))
    hp = h_positions.astype(jnp.int32).reshape(1, NPOS)
    vp = v_positions.astype(jnp.int32).reshape(1, NPOS)

    means, ohct, ridx = pl.pallas_call(
        _reduce_kernel,
        grid=(H // HB,),
        in_specs=[
            pl.BlockSpec(memory_space=pltpu.SMEM),
            pl.BlockSpec(memory_space=pltpu.SMEM),
            pl.BlockSpec((HB, C, W), lambda h: (h, 0, 0)),
        ],
        out_specs=[
            pl.BlockSpec((NSEG, C, NSEG), lambda h: (0, 0, 0)),
            pl.BlockSpec((NSEG, W), lambda h: (0, 0)),
            pl.BlockSpec((HB, 1), lambda h: (h, 0)),
        ],
        out_shape=[
            jax.ShapeDtypeStruct((NSEG, C, NSEG), jnp.float32),
            jax.ShapeDtypeStruct((NSEG, W), jnp.float32),
            jax.ShapeDtypeStruct((H, 1), jnp.int32),
        ],
        scratch_shapes=[pltpu.VMEM((NSEG, C, W), jnp.float32)],
    )(hp, vp, xt)

    yt = pl.pallas_call(
        _expand_kernel,
        grid=(H // IB,),
        in_specs=[
            pl.BlockSpec(memory_space=pltpu.SMEM),
            pl.BlockSpec(memory_space=pltpu.SMEM),
            pl.BlockSpec((NSEG, C, NSEG), lambda h: (0, 0, 0)),
            pl.BlockSpec((NSEG, W), lambda h: (0, 0)),
        ],
        out_specs=pl.BlockSpec((IB, C, W), lambda h: (h, 0, 0)),
        out_shape=jax.ShapeDtypeStruct((H, C, W), jnp.float32),
    )(hp, ridx, means, ohct)

    return jnp.transpose(yt, (0, 2, 1))[None]
